# baseline (speedup 1.0000x reference)
import functools
import math

import jax
import jax.numpy as jnp
from jax import lax
from jax.experimental import pallas as pl
from jax.experimental.pallas import tpu as pltpu
from jax.experimental.pallas import tpu_sc as plsc

B, S, D, E, FF, TOP_K = 2, 2048, 2048, 8, 8192, 2
T = B * S
A = T * TOP_K

TM = 512
NT = A // TM + E
P = NT * TM
FFB = 1024
NF = FF // FFB

TB = 512
_SQRT2 = math.sqrt(2.0)

_NC, _NS = 2, 16
NW = _NC * _NS
DW = D // 2
G_CH = 48
C_CH = 16


def _router_body(x_ref, wg_ref, lg_ref, idx_ref, w_ref):
    xb = x_ref[...]
    lg = lax.dot_general(xb, wg_ref[...], (((1,), (1,)), ((), ())),
                         preferred_element_type=jnp.float32)
    lg_ref[...] = lg
    m = jnp.max(lg, axis=-1, keepdims=True)
    p = jnp.exp(lg - m)
    p = p / jnp.sum(p, axis=-1, keepdims=True)
    iota8 = lax.broadcasted_iota(jnp.int32, (TB, E), 1)
    m1 = jnp.max(p, axis=-1, keepdims=True)
    a1 = jnp.min(jnp.where(p == m1, iota8, E), axis=-1, keepdims=True)
    p2 = jnp.where(iota8 == a1, -jnp.inf, p)
    m2 = jnp.max(p2, axis=-1, keepdims=True)
    a2 = jnp.min(jnp.where(p2 == m2, iota8, E), axis=-1, keepdims=True)
    denom = m1 + m2
    idx_ref[...] = jnp.concatenate([a1, a2], axis=1)
    w_ref[...] = jnp.concatenate([m1 / denom, m2 / denom], axis=1)


def _router(xf, Wg):
    return pl.pallas_call(
        _router_body,
        grid=(T // TB,),
        in_specs=[
            pl.BlockSpec((TB, D), lambda t: (t, 0)),
            pl.BlockSpec((E, D), lambda t: (0, 0)),
        ],
        out_specs=[
            pl.BlockSpec((TB, E), lambda t: (t, 0)),
            pl.BlockSpec((TB, TOP_K), lambda t: (t, 0)),
            pl.BlockSpec((TB, TOP_K), lambda t: (t, 0)),
        ],
        out_shape=[
            jax.ShapeDtypeStruct((T, E), jnp.float32),
            jax.ShapeDtypeStruct((T, TOP_K), jnp.int32),
            jax.ShapeDtypeStruct((T, TOP_K), jnp.float32),
        ],
    )(xf, Wg)


def _group(idx, w):
    e_flat = idx.reshape(-1)
    oh = (e_flat[:, None] == jnp.arange(E, dtype=jnp.int32)[None, :]).astype(jnp.int32)
    csum = jnp.cumsum(oh, axis=0)
    counts = csum[-1]
    rank = jnp.take_along_axis(csum, e_flat[:, None], axis=1)[:, 0] - 1
    padded = ((counts + TM - 1) // TM) * TM
    pend = jnp.cumsum(padded)
    pstart = pend - padded
    dest = pstart[e_flat] + rank
    tok = jnp.arange(A, dtype=jnp.int32) // TOP_K
    src_row = jnp.zeros((P,), jnp.int32).at[dest].set(tok)
    wpos = jnp.zeros((P,), jnp.float32).at[dest].set(w.reshape(-1))
    tile_start = jnp.arange(NT, dtype=jnp.int32) * TM
    te = jnp.sum((tile_start[:, None] >= pend[None, :]).astype(jnp.int32), axis=1)
    te = jnp.minimum(te, E - 1).astype(jnp.int32)
    nt_used = (pend[-1] // TM).astype(jnp.int32)
    te = jnp.concatenate([te, nt_used[None]])
    return src_row, wpos, te, dest.astype(jnp.int32)


_G_PER_W = P // NW
_G_ITERS = _G_PER_W // G_CH


@functools.cache
def _sc_mesh():
    return plsc.VectorSubcoreMesh(core_axis_name="c", subcore_axis_name="s")


@functools.cache
def _make_sc_gather():
    @functools.partial(
        pl.kernel,
        mesh=_sc_mesh(),
        out_type=jax.ShapeDtypeStruct((P, DW), jnp.int32),
        scratch_types=[
            pltpu.VMEM((_G_PER_W,), jnp.int32),
            pltpu.VMEM((G_CH, DW), jnp.int32),
            pltpu.VMEM((G_CH, DW), jnp.int32),
            pltpu.SemaphoreType.DMA,
            pltpu.SemaphoreType.DMA,
            pltpu.SemaphoreType.DMA,
            pltpu.SemaphoreType.DMA,
        ],
    )
    def _sc_gather(x_hbm, idx_hbm, out_hbm, idx_all, r0, r1, g0, g1, s0, s1):
        wid = lax.axis_index("s") * _NC + lax.axis_index("c")
        base = wid * _G_PER_W
        pltpu.sync_copy(idx_hbm.at[pl.ds(base, _G_PER_W)], idx_all)
        rows, gsem, ssem = (r0, r1), (g0, g1), (s0, s1)

        def gstart(i, b):
            return pltpu.async_copy(
                x_hbm.at[idx_all.at[pl.ds(i * G_CH, G_CH)]], rows[b], gsem[b])

        def sstart(i, b):
            return pltpu.async_copy(
                rows[b], out_hbm.at[pl.ds(base + i * G_CH, G_CH)], ssem[b])

        g = gstart(0, 0)
        stores = [None, None]
        for i in range(_G_ITERS):
            b = i & 1
            g.wait()
            stores[b] = sstart(i, b)
            if i + 1 < _G_ITERS:
                nb = 1 - b
                if stores[nb] is not None:
                    stores[nb].wait()
                    stores[nb] = None
                g = gstart(i + 1, nb)
        for st in stores:
            if st is not None:
                st.wait()

    return _sc_gather


def _ffn_body(te_ref, xg_ref, w1_ref, b1_ref, w2_ref, b2_ref, wp_ref,
              og_ref, xs_ref, acc_ref):
    t = pl.program_id(0)
    f = pl.program_id(1)

    @pl.when(t < te_ref[NT])
    def _():
        @pl.when(f == 0)
        def _():
            xs_ref[...] = xg_ref[...].astype(jnp.float32)

        xb = xs_ref[...]
        h = lax.dot_general(xb, w1_ref[0], (((1,), (1,)), ((), ())),
                            preferred_element_type=jnp.float32)
        h = h + b1_ref[0]
        h = 0.5 * h * (1.0 + lax.erf(h / _SQRT2))
        contrib = lax.dot_general(h, w2_ref[0], (((1,), (1,)), ((), ())),
                                  preferred_element_type=jnp.float32)

        @pl.when(f == 0)
        def _():
            acc_ref[...] = contrib

        @pl.when(f > 0)
        def _():
            acc_ref[...] = acc_ref[...] + contrib

        @pl.when(f == NF - 1)
        def _():
            og_ref[...] = wp_ref[...] * (acc_ref[...] + b2_ref[0])


def _ffn(te, xg, W1, b1, W2, b2, wpos):
    b1r = b1.reshape(E * NF, 1, FFB)
    b2r = b2.reshape(E, 1, D)
    wp = wpos.reshape(P, 1)

    def _tc(t, te_r):
        return jnp.minimum(t, te_r[NT] - 1)

    grid_spec = pltpu.PrefetchScalarGridSpec(
        num_scalar_prefetch=1,
        grid=(NT, NF),
        in_specs=[
            pl.BlockSpec((TM, D), lambda t, f, te_r: (_tc(t, te_r), 0)),
            pl.BlockSpec((1, FFB, D),
                         lambda t, f, te_r: (te_r[_tc(t, te_r)], f, 0)),
            pl.BlockSpec((1, 1, FFB),
                         lambda t, f, te_r: (te_r[_tc(t, te_r)] * NF + f, 0, 0)),
            pl.BlockSpec((1, D, FFB),
                         lambda t, f, te_r: (te_r[_tc(t, te_r)], 0, f)),
            pl.BlockSpec((1, 1, D),
                         lambda t, f, te_r: (te_r[_tc(t, te_r)], 0, 0)),
            pl.BlockSpec((TM, 1), lambda t, f, te_r: (_tc(t, te_r), 0)),
        ],
        out_specs=pl.BlockSpec((TM, D), lambda t, f, te_r: (_tc(t, te_r), 0)),
        scratch_shapes=[pltpu.VMEM((TM, D), jnp.float32),
                        pltpu.VMEM((TM, D), jnp.float32)],
    )
    return pl.pallas_call(
        _ffn_body,
        grid_spec=grid_spec,
        out_shape=jax.ShapeDtypeStruct((P, D), jnp.float32),
        compiler_params=pltpu.CompilerParams(
            dimension_semantics=("arbitrary", "arbitrary")),
    )(te, xg, W1, b1r, W2, b2r, wp)


_C_IN = 16
_C_OUT = _C_IN // 2
_C2_PER_W = A // NW
_C2_ITERS = _C2_PER_W // _C_IN


@functools.cache
def _make_sc_combine():
    @functools.partial(
        pl.kernel,
        mesh=_sc_mesh(),
        out_type=jax.ShapeDtypeStruct((T, D), jnp.float32),
        scratch_types=[
            pltpu.VMEM((_C2_PER_W,), jnp.int32),
            pltpu.VMEM((_C_IN, D), jnp.float32),
            pltpu.VMEM((_C_IN, D), jnp.float32),
            pltpu.VMEM((_C_OUT, D), jnp.float32),
            pltpu.VMEM((_C_OUT, D), jnp.float32),
            pltpu.SemaphoreType.DMA,
            pltpu.SemaphoreType.DMA,
            pltpu.SemaphoreType.DMA,
            pltpu.SemaphoreType.DMA,
        ],
    )
    def _sc_combine(og_hbm, dest_hbm, out_hbm,
                    idx_all, in0, in1, o0, o1, g0, g1, s0, s1):
        wid = lax.axis_index("s") * _NC + lax.axis_index("c")
        rbase = wid * _C2_PER_W
        tbase = wid * (_C2_PER_W // 2)
        pltpu.sync_copy(dest_hbm.at[pl.ds(rbase, _C2_PER_W)], idx_all)
        ins, outs = (in0, in1), (o0, o1)
        gsem, ssem = (g0, g1), (s0, s1)

        def gstart(i, b):
            return pltpu.async_copy(
                og_hbm.at[idx_all.at[pl.ds(i * _C_IN, _C_IN)]], ins[b], gsem[b])

        def sstart(i, b):
            return pltpu.async_copy(
                outs[b], out_hbm.at[pl.ds(tbase + i * _C_OUT, _C_OUT)], ssem[b])

        g = gstart(0, 0)
        stores = [None, None]
        for i in range(_C2_ITERS):
            b = i & 1
            g.wait()
            if i + 1 < _C2_ITERS:
                g = gstart(i + 1, 1 - b)
            if stores[b] is not None:
                stores[b].wait()
                stores[b] = None

            def add_col(j, c, _b=b):
                for k in range(_C_OUT):
                    sl = pl.ds(j * 16, 16)
                    outs[_b][k, sl] = ins[_b][2 * k, sl] + ins[_b][2 * k + 1, sl]
                return c

            lax.fori_loop(0, D // 16, add_col, 0)
            stores[b] = sstart(i, b)
        for st in stores:
            if st is not None:
                st.wait()

    return _sc_combine


def kernel(x, Wg, W1, b1, W2, b2):
    b, s, d = x.shape
    xf = x.reshape(-1, d)
    logits, idx, w = _router(xf, Wg)
    src_row, wpos, te, dest = _group(idx, w)
    xp = lax.bitcast_convert_type(
        x.astype(jnp.bfloat16).reshape(T, D // 2, 2), jnp.int32)
    xgp = _make_sc_gather()(xp, src_row)
    xg = lax.bitcast_convert_type(xgp, jnp.bfloat16).reshape(P, D)
    og = _ffn(te, xg, W1, b1, W2, b2, wpos)
    out = _make_sc_combine()(og, dest)
    return out.reshape(b, s, d), logits


# speedup vs baseline: 1.4022x; 1.4022x over previous
import functools
import math

import jax
import jax.numpy as jnp
from jax import lax
from jax.experimental import pallas as pl
from jax.experimental.pallas import tpu as pltpu
from jax.experimental.pallas import tpu_sc as plsc

B, S, D, E, FF, TOP_K = 2, 2048, 2048, 8, 8192, 2
T = B * S
A = T * TOP_K

TM = 512
NT = A // TM + E
P = NT * TM
FFB = 1024
NF = FF // FFB

TB = 512
_SQRT2 = math.sqrt(2.0)

_NC, _NS = 2, 16
NW = _NC * _NS
G_CH = 24
C_CH = 16


def _router_body(x_ref, wg_ref, lg_ref, idx_ref, w_ref):
    xb = x_ref[...]
    lg = lax.dot_general(xb, wg_ref[...], (((1,), (1,)), ((), ())),
                         preferred_element_type=jnp.float32)
    lg_ref[...] = lg
    m = jnp.max(lg, axis=-1, keepdims=True)
    p = jnp.exp(lg - m)
    p = p / jnp.sum(p, axis=-1, keepdims=True)
    iota8 = lax.broadcasted_iota(jnp.int32, (TB, E), 1)
    m1 = jnp.max(p, axis=-1, keepdims=True)
    a1 = jnp.min(jnp.where(p == m1, iota8, E), axis=-1, keepdims=True)
    p2 = jnp.where(iota8 == a1, -jnp.inf, p)
    m2 = jnp.max(p2, axis=-1, keepdims=True)
    a2 = jnp.min(jnp.where(p2 == m2, iota8, E), axis=-1, keepdims=True)
    denom = m1 + m2
    idx_ref[...] = jnp.concatenate([a1, a2], axis=1)
    w_ref[...] = jnp.concatenate([m1 / denom, m2 / denom], axis=1)


def _router(xf, Wg):
    return pl.pallas_call(
        _router_body,
        grid=(T // TB,),
        in_specs=[
            pl.BlockSpec((TB, D), lambda t: (t, 0)),
            pl.BlockSpec((E, D), lambda t: (0, 0)),
        ],
        out_specs=[
            pl.BlockSpec((TB, E), lambda t: (t, 0)),
            pl.BlockSpec((TB, TOP_K), lambda t: (t, 0)),
            pl.BlockSpec((TB, TOP_K), lambda t: (t, 0)),
        ],
        out_shape=[
            jax.ShapeDtypeStruct((T, E), jnp.float32),
            jax.ShapeDtypeStruct((T, TOP_K), jnp.int32),
            jax.ShapeDtypeStruct((T, TOP_K), jnp.float32),
        ],
    )(xf, Wg)


def _group(idx, w):
    e_flat = idx.reshape(-1)
    oh = (e_flat[:, None] == jnp.arange(E, dtype=jnp.int32)[None, :]).astype(jnp.int32)
    csum = jnp.cumsum(oh, axis=0)
    counts = csum[-1]
    rank = jnp.take_along_axis(csum, e_flat[:, None], axis=1)[:, 0] - 1
    padded = ((counts + TM - 1) // TM) * TM
    pend = jnp.cumsum(padded)
    pstart = pend - padded
    dest = pstart[e_flat] + rank
    tok = jnp.arange(A, dtype=jnp.int32) // TOP_K
    src_row = jnp.zeros((P,), jnp.int32).at[dest].set(tok)
    wpos = jnp.zeros((P,), jnp.float32).at[dest].set(w.reshape(-1))
    tile_start = jnp.arange(NT, dtype=jnp.int32) * TM
    te = jnp.sum((tile_start[:, None] >= pend[None, :]).astype(jnp.int32), axis=1)
    te = jnp.minimum(te, E - 1).astype(jnp.int32)
    nt_used = (pend[-1] // TM).astype(jnp.int32)
    te = jnp.concatenate([te, nt_used[None]])
    return src_row, wpos, te, dest.astype(jnp.int32)


_G_PER_W = P // NW
_G_ITERS = _G_PER_W // G_CH


@functools.cache
def _sc_mesh():
    return plsc.VectorSubcoreMesh(core_axis_name="c", subcore_axis_name="s")


@functools.cache
def _make_sc_gather():
    @functools.partial(
        pl.kernel,
        mesh=_sc_mesh(),
        out_type=jax.ShapeDtypeStruct((P, D), jnp.float32),
        scratch_types=[
            pltpu.VMEM((_G_PER_W,), jnp.int32),
            pltpu.VMEM((G_CH, D), jnp.float32),
            pltpu.VMEM((G_CH, D), jnp.float32),
            pltpu.SemaphoreType.DMA,
            pltpu.SemaphoreType.DMA,
            pltpu.SemaphoreType.DMA,
            pltpu.SemaphoreType.DMA,
        ],
    )
    def _sc_gather(x_hbm, idx_hbm, out_hbm, idx_all, r0, r1, g0, g1, s0, s1):
        wid = lax.axis_index("s") * _NC + lax.axis_index("c")
        base = wid * _G_PER_W
        pltpu.sync_copy(idx_hbm.at[pl.ds(base, _G_PER_W)], idx_all)
        rows, gsem, ssem = (r0, r1), (g0, g1), (s0, s1)

        def gstart(i, b):
            return pltpu.async_copy(
                x_hbm.at[idx_all.at[pl.ds(i * G_CH, G_CH)]], rows[b], gsem[b])

        def sstart(i, b):
            return pltpu.async_copy(
                rows[b], out_hbm.at[pl.ds(base + i * G_CH, G_CH)], ssem[b])

        g = gstart(0, 0)
        stores = [None, None]
        for i in range(_G_ITERS):
            b = i & 1
            g.wait()
            stores[b] = sstart(i, b)
            if i + 1 < _G_ITERS:
                nb = 1 - b
                if stores[nb] is not None:
                    stores[nb].wait()
                    stores[nb] = None
                g = gstart(i + 1, nb)
        for st in stores:
            if st is not None:
                st.wait()

    return _sc_gather


def _ffn_body(te_ref, xg_ref, w1_ref, b1_ref, w2_ref, b2_ref, wp_ref,
              og_ref, acc_ref):
    t = pl.program_id(0)
    f = pl.program_id(1)

    @pl.when(t < te_ref[NT])
    def _():
        xb = xg_ref[...]
        h = lax.dot_general(xb, w1_ref[0], (((1,), (1,)), ((), ())),
                            preferred_element_type=jnp.float32)
        h = h + b1_ref[0]
        h = 0.5 * h * (1.0 + lax.erf(h / _SQRT2))
        contrib = lax.dot_general(h, w2_ref[0], (((1,), (1,)), ((), ())),
                                  preferred_element_type=jnp.float32)

        @pl.when(f == 0)
        def _():
            acc_ref[...] = contrib

        @pl.when(f > 0)
        def _():
            acc_ref[...] = acc_ref[...] + contrib

        @pl.when(f == NF - 1)
        def _():
            og_ref[...] = wp_ref[...] * (acc_ref[...] + b2_ref[0])


def _ffn(te, xg, W1, b1, W2, b2, wpos):
    b1r = b1.reshape(E * NF, 1, FFB)
    b2r = b2.reshape(E, 1, D)
    wp = wpos.reshape(P, 1)

    def _tc(t, te_r):
        return jnp.minimum(t, te_r[NT] - 1)

    grid_spec = pltpu.PrefetchScalarGridSpec(
        num_scalar_prefetch=1,
        grid=(NT, NF),
        in_specs=[
            pl.BlockSpec((TM, D), lambda t, f, te_r: (_tc(t, te_r), 0)),
            pl.BlockSpec((1, FFB, D),
                         lambda t, f, te_r: (te_r[_tc(t, te_r)], f, 0)),
            pl.BlockSpec((1, 1, FFB),
                         lambda t, f, te_r: (te_r[_tc(t, te_r)] * NF + f, 0, 0)),
            pl.BlockSpec((1, D, FFB),
                         lambda t, f, te_r: (te_r[_tc(t, te_r)], 0, f)),
            pl.BlockSpec((1, 1, D),
                         lambda t, f, te_r: (te_r[_tc(t, te_r)], 0, 0)),
            pl.BlockSpec((TM, 1), lambda t, f, te_r: (_tc(t, te_r), 0)),
        ],
        out_specs=pl.BlockSpec((TM, D), lambda t, f, te_r: (_tc(t, te_r), 0)),
        scratch_shapes=[pltpu.VMEM((TM, D), jnp.float32)],
    )
    return pl.pallas_call(
        _ffn_body,
        grid_spec=grid_spec,
        out_shape=jax.ShapeDtypeStruct((P, D), jnp.float32),
        compiler_params=pltpu.CompilerParams(
            dimension_semantics=("arbitrary", "arbitrary")),
    )(te, xg, W1, b1r, W2, b2r, wp)


_C_IN = 16
_C_OUT = _C_IN // 2
_C2_PER_W = A // NW
_C2_ITERS = _C2_PER_W // _C_IN


@functools.cache
def _make_sc_combine():
    @functools.partial(
        pl.kernel,
        mesh=_sc_mesh(),
        out_type=jax.ShapeDtypeStruct((T, D), jnp.float32),
        scratch_types=[
            pltpu.VMEM((_C2_PER_W,), jnp.int32),
            pltpu.VMEM((_C_IN, D), jnp.float32),
            pltpu.VMEM((_C_IN, D), jnp.float32),
            pltpu.VMEM((_C_OUT, D), jnp.float32),
            pltpu.VMEM((_C_OUT, D), jnp.float32),
            pltpu.SemaphoreType.DMA,
            pltpu.SemaphoreType.DMA,
            pltpu.SemaphoreType.DMA,
            pltpu.SemaphoreType.DMA,
        ],
    )
    def _sc_combine(og_hbm, dest_hbm, out_hbm,
                    idx_all, in0, in1, o0, o1, g0, g1, s0, s1):
        wid = lax.axis_index("s") * _NC + lax.axis_index("c")
        rbase = wid * _C2_PER_W
        tbase = wid * (_C2_PER_W // 2)
        pltpu.sync_copy(dest_hbm.at[pl.ds(rbase, _C2_PER_W)], idx_all)
        ins, outs = (in0, in1), (o0, o1)
        gsem, ssem = (g0, g1), (s0, s1)

        def gstart(i, b):
            return pltpu.async_copy(
                og_hbm.at[idx_all.at[pl.ds(i * _C_IN, _C_IN)]], ins[b], gsem[b])

        def sstart(i, b):
            return pltpu.async_copy(
                outs[b], out_hbm.at[pl.ds(tbase + i * _C_OUT, _C_OUT)], ssem[b])

        g = gstart(0, 0)
        stores = [None, None]
        for i in range(_C2_ITERS):
            b = i & 1
            g.wait()
            if i + 1 < _C2_ITERS:
                g = gstart(i + 1, 1 - b)
            if stores[b] is not None:
                stores[b].wait()
                stores[b] = None

            def add_col(j, c, _b=b):
                for k in range(_C_OUT):
                    sl = pl.ds(j * 16, 16)
                    outs[_b][k, sl] = ins[_b][2 * k, sl] + ins[_b][2 * k + 1, sl]
                return c

            lax.fori_loop(0, D // 16, add_col, 0)
            stores[b] = sstart(i, b)
        for st in stores:
            if st is not None:
                st.wait()

    return _sc_combine


def kernel(x, Wg, W1, b1, W2, b2):
    b, s, d = x.shape
    xf = x.reshape(-1, d)
    logits, idx, w = _router(xf, Wg)
    src_row, wpos, te, dest = _group(idx, w)
    xg = _make_sc_gather()(xf, src_row)
    og = _ffn(te, xg, W1, b1, W2, b2, wpos)
    out = _make_sc_combine()(og, dest)
    return out.reshape(b, s, d), logits
